# baseline (speedup 1.0000x reference)
import functools

import jax
import jax.numpy as jnp
from jax import lax
from jax.experimental import pallas as pl
from jax.experimental.pallas import tpu as pltpu
from jax.experimental.pallas import tpu_sc as plsc

_K = 5
_NC = 2
_NS = 16
_NW = _NC * _NS
_L = 16
_CHUNK = 32


def _insert(ts, x):
    out = []
    for t in ts[:-1]:
        out.append(jnp.maximum(t, x))
        x = jnp.minimum(t, x)
    out.append(jnp.maximum(ts[-1], x))
    return out


def kernel(s, y, m_list, Z):
    B, D = s.shape
    S = Z.shape[2]
    sf = s.reshape(B * D)
    zf = Z.reshape(B * D * S)
    rows_per_w = B // _NW
    n_chunks = rows_per_w // _CHUNK

    mesh = plsc.VectorSubcoreMesh(core_axis_name="c", subcore_axis_name="s",
                                  num_cores=_NC, num_subcores=_NS)

    @functools.partial(
        pl.kernel,
        out_type=jax.ShapeDtypeStruct((_NW, _L), jnp.float32),
        mesh=mesh,
        scratch_types=[
            pltpu.VMEM((_CHUNK * D,), jnp.float32),
            pltpu.VMEM((_CHUNK,), jnp.int32),
            pltpu.VMEM((D,), jnp.float32),
            pltpu.VMEM((_CHUNK * D * S,), jnp.float32),
            pltpu.VMEM((_L,), jnp.float32),
        ],
        compiler_params=pltpu.CompilerParams(needs_layout_passes=False),
    )
    def sc_kernel(s_hbm, y_hbm, m_hbm, z_hbm, out_hbm, s_v, y_v, m_v, z_v,
                  acc_v):
        cid = lax.axis_index("c")
        sid = lax.axis_index("s")
        wid = sid * _NC + cid
        w_base = wid * rows_per_w
        iota = lax.iota(jnp.int32, _L)
        pltpu.sync_copy(m_hbm, m_v)

        def chunk_body(ci, acc):
            base = w_base + ci * _CHUNK
            pltpu.sync_copy(s_hbm.at[pl.ds(base * D, _CHUNK * D)], s_v)
            pltpu.sync_copy(y_hbm.at[pl.ds(base, _CHUNK)], y_v)
            pltpu.sync_copy(z_hbm.at[pl.ds(base * D * S, _CHUNK * D * S)], z_v)
            for g in range(_CHUNK // _L):
                ridx = iota + (g * _L)
                sidx = ridx * D
                zidx = ridx * (D * S)
                y_vec = y_v[pl.ds(g * _L, _L)]

                def dbody(d, ts):
                    sval = plsc.load_gather(s_v, [sidx + d])
                    basev = jnp.where(y_vec == d, jnp.float32(-1e10),
                                      sval * jnp.float32(500.0))
                    new = []
                    for j in range(S):
                        z = plsc.load_gather(z_v, [zidx + (d * S + j)])
                        new.extend(_insert(list(ts[j * _K:(j + 1) * _K]),
                                           basev + z))
                    return tuple(new)

                init = tuple(jnp.full((_L,), -3e38, jnp.float32)
                             for _ in range(_K * S))
                ts = lax.fori_loop(0, D, dbody, init)
                sum5 = (ts[_K - 1] + ts[2 * _K - 1] + ts[3 * _K - 1]
                        + ts[4 * _K - 1] + ts[5 * _K - 1])
                sigma = sum5 * jnp.float32(1.0 / (10.0 * S))
                sg = plsc.load_gather(s_v, [sidx + y_vec])
                mg = plsc.load_gather(m_v, [y_vec])
                s_y = (sg - mg) * jnp.float32(50.0)
                loss = jnp.maximum(jnp.float32(1.0) + sigma - s_y,
                                   jnp.float32(0.0))
                acc = acc + loss
            return acc

        acc = lax.fori_loop(0, n_chunks, chunk_body,
                            jnp.zeros((_L,), jnp.float32))
        acc_v[...] = acc
        pltpu.sync_copy(acc_v, out_hbm.at[wid])

    partials = sc_kernel(sf, y, m_list, zf)
    return jnp.sum(partials) / jnp.float32(B)


# speedup vs baseline: 12.1159x; 12.1159x over previous
import functools

import jax
import jax.numpy as jnp
from jax import lax
from jax.experimental import pallas as pl
from jax.experimental.pallas import tpu as pltpu
from jax.experimental.pallas import tpu_sc as plsc

_K = 5
_NC = 2
_NS = 16
_NW = _NC * _NS
_L = 16
_LANES = 128


def _insert(ts, x):
    out = []
    for t in ts[:-1]:
        out.append(jnp.maximum(t, x))
        x = jnp.minimum(t, x)
    out.append(jnp.maximum(ts[-1], x))
    return out


def kernel(s, y, m_list, Z):
    B, D = s.shape
    S = Z.shape[2]
    sT = jnp.transpose(s)
    zT = jnp.transpose(Z, (2, 1, 0))
    n_stripes = B // _LANES
    stripes_per_w = n_stripes // _NW

    mesh = plsc.VectorSubcoreMesh(core_axis_name="c", subcore_axis_name="s",
                                  num_cores=_NC, num_subcores=_NS)

    @functools.partial(
        pl.kernel,
        out_type=jax.ShapeDtypeStruct((_NW, _L), jnp.float32),
        mesh=mesh,
        scratch_types=[
            pltpu.VMEM((D, _LANES), jnp.float32),
            pltpu.VMEM((_LANES,), jnp.int32),
            pltpu.VMEM((D,), jnp.float32),
        ] + [pltpu.VMEM((D, _LANES), jnp.float32) for _ in range(5)] + [
            pltpu.VMEM((_L,), jnp.float32),
        ],
        compiler_params=pltpu.CompilerParams(needs_layout_passes=False,
                                             use_tc_tiling_on_sc=True),
    )
    def sc_kernel(s_hbm, y_hbm, m_hbm, z_hbm, out_hbm, s_v, y_v, m_v,
                  z0_v, z1_v, z2_v, z3_v, z4_v, acc_v):
        z_bufs = (z0_v, z1_v, z2_v, z3_v, z4_v)
        cid = lax.axis_index("c")
        sid = lax.axis_index("s")
        wid = sid * _NC + cid
        pltpu.sync_copy(m_hbm, m_v)

        def stripe_body(si, acc):
            b0 = (wid * stripes_per_w + si) * _LANES
            pltpu.sync_copy(s_hbm.at[:, pl.ds(b0, _LANES)], s_v)
            pltpu.sync_copy(y_hbm.at[pl.ds(b0, _LANES)], y_v)
            for j in range(S):
                pltpu.sync_copy(z_hbm.at[j, :, pl.ds(b0, _LANES)], z_bufs[j])
            for lg in range(_LANES // _L):
                y_vec = y_v[pl.ds(lg * _L, _L)]

                def dbody(d, carry):
                    ts = carry[:-1]
                    sy = carry[-1]
                    sval = s_v[d, pl.ds(lg * _L, _L)]
                    eq = y_vec == d
                    basev = jnp.where(eq, jnp.float32(-1e10),
                                      sval * jnp.float32(500.0))
                    sy = sy + jnp.where(eq, sval, jnp.float32(0.0))
                    new = []
                    for j in range(S):
                        z = z_bufs[j][d, pl.ds(lg * _L, _L)]
                        new.extend(_insert(list(ts[j * _K:(j + 1) * _K]),
                                           basev + z))
                    new.append(sy)
                    return tuple(new)

                init = tuple(jnp.full((_L,), -3e38, jnp.float32)
                             for _ in range(_K * S))
                init = init + (jnp.zeros((_L,), jnp.float32),)
                res = lax.fori_loop(0, D, dbody, init)
                ts, sy = res[:-1], res[-1]
                sum5 = (ts[_K - 1] + ts[2 * _K - 1] + ts[3 * _K - 1]
                        + ts[4 * _K - 1] + ts[5 * _K - 1])
                sigma = sum5 * jnp.float32(1.0 / (10.0 * S))
                mg = plsc.load_gather(m_v, [y_vec])
                s_y = (sy - mg) * jnp.float32(50.0)
                loss = jnp.maximum(jnp.float32(1.0) + sigma - s_y,
                                   jnp.float32(0.0))
                acc = acc + loss
            return acc

        acc = lax.fori_loop(0, stripes_per_w, stripe_body,
                            jnp.zeros((_L,), jnp.float32))
        acc_v[...] = acc
        pltpu.sync_copy(acc_v, out_hbm.at[wid])

    partials = sc_kernel(sT, y, m_list, zT)
    return jnp.sum(partials) / jnp.float32(B)


# speedup vs baseline: 14.3245x; 1.1823x over previous
import functools

import jax
import jax.numpy as jnp
from jax import lax
from jax.experimental import pallas as pl
from jax.experimental.pallas import tpu as pltpu
from jax.experimental.pallas import tpu_sc as plsc

_K = 5
_NC = 2
_NS = 16
_NW = _NC * _NS
_L = 16
_LANES = 128
_NSTRIPES = 4
_NDB = 3


def _insert(ts, x):
    out = []
    for t in ts[:-1]:
        out.append(jnp.maximum(t, x))
        x = jnp.minimum(t, x)
    out.append(jnp.maximum(ts[-1], x))
    return out


def kernel(s, y, m_list, Z):
    B, D = s.shape
    S = Z.shape[2]
    sT = jnp.transpose(s)
    zT = jnp.transpose(Z, (2, 1, 0))
    rows_per_w = B // _NW

    mesh = plsc.VectorSubcoreMesh(core_axis_name="c", subcore_axis_name="s",
                                  num_cores=_NC, num_subcores=_NS)

    @functools.partial(
        pl.kernel,
        out_type=jax.ShapeDtypeStruct((_NW, _L), jnp.float32),
        mesh=mesh,
        scratch_types=[
            pltpu.VMEM((D,), jnp.float32),
            pltpu.VMEM((_L,), jnp.float32),
            pltpu.VMEM((D, _LANES), jnp.float32),
            pltpu.VMEM((_LANES,), jnp.int32),
            pltpu.VMEM((D, _LANES), jnp.float32),
            pltpu.VMEM((D, _LANES), jnp.float32),
        ] + [pltpu.VMEM((D, _LANES), jnp.float32) for _ in range(2 * _NDB)]
          + [pltpu.SemaphoreType.DMA] * 3,
        compiler_params=pltpu.CompilerParams(needs_layout_passes=False,
                                             use_tc_tiling_on_sc=True),
    )
    def sc_kernel(s_hbm, y_hbm, m_hbm, z_hbm, out_hbm, m_v, acc_v,
                  s_v, y_v, z3_v, z4_v,
                  za0, zb0, zc0, za1, zb1, zc1,
                  sem_a, sem_b, sem_s):
        slots = ((za0, zb0, zc0, sem_a), (za1, zb1, zc1, sem_b))
        cid = lax.axis_index("c")
        sid = lax.axis_index("s")
        wid = sid * _NC + cid
        w_base = wid * rows_per_w
        pltpu.sync_copy(m_hbm, m_v)

        def z03_copies(si, slot):
            b0 = w_base + si * _LANES
            sem = slot[3]
            return [pltpu.make_async_copy(z_hbm.at[j, :, pl.ds(b0, _LANES)],
                                          slot[j], sem) for j in range(_NDB)]

        def rest_copies(si):
            b0 = w_base + si * _LANES
            return [
                pltpu.make_async_copy(s_hbm.at[:, pl.ds(b0, _LANES)], s_v,
                                      sem_s),
                pltpu.make_async_copy(y_hbm.at[pl.ds(b0, _LANES)], y_v, sem_s),
                pltpu.make_async_copy(z_hbm.at[3, :, pl.ds(b0, _LANES)],
                                      z3_v, sem_s),
                pltpu.make_async_copy(z_hbm.at[4, :, pl.ds(b0, _LANES)],
                                      z4_v, sem_s),
            ]

        def compute_stripe(slot, acc):
            z_vs = (slot[0], slot[1], slot[2], z3_v, z4_v)

            def lg_body(lg, acc):
                l0 = lg * _L
                y_vec = y_v[pl.ds(l0, _L)]

                def dbody(d, carry):
                    ts = carry[:-1]
                    sy = carry[-1]
                    sval = s_v[d, pl.ds(l0, _L)]
                    eq = y_vec == d
                    basev = jnp.where(eq, jnp.float32(-1e10),
                                      sval * jnp.float32(500.0))
                    sy = sy + jnp.where(eq, sval, jnp.float32(0.0))
                    new = []
                    for j in range(S):
                        z = z_vs[j][d, pl.ds(l0, _L)]
                        new.extend(_insert(list(ts[j * _K:(j + 1) * _K]),
                                           basev + z))
                    new.append(sy)
                    return tuple(new)

                init = tuple(jnp.full((_L,), -3e38, jnp.float32)
                             for _ in range(_K * S))
                init = init + (jnp.zeros((_L,), jnp.float32),)
                res = lax.fori_loop(0, D, dbody, init)
                ts, sy = res[:-1], res[-1]
                sum5 = (ts[_K - 1] + ts[2 * _K - 1] + ts[3 * _K - 1]
                        + ts[4 * _K - 1] + ts[5 * _K - 1])
                sigma = sum5 * jnp.float32(1.0 / (10.0 * S))
                mg = plsc.load_gather(m_v, [y_vec])
                s_y = (sy - mg) * jnp.float32(50.0)
                loss = jnp.maximum(jnp.float32(1.0) + sigma - s_y,
                                   jnp.float32(0.0))
                return acc + loss

            return lax.fori_loop(0, _LANES // _L, lg_body, acc)

        acc = jnp.zeros((_L,), jnp.float32)
        for c in z03_copies(0, slots[0]):
            c.start()
        for si in range(_NSTRIPES):
            slot = slots[si % 2]
            rest = rest_copies(si)
            for c in rest:
                c.start()
            if si + 1 < _NSTRIPES:
                for c in z03_copies(si + 1, slots[(si + 1) % 2]):
                    c.start()
            for c in rest:
                c.wait()
            for c in z03_copies(si, slot):
                c.wait()
            acc = compute_stripe(slot, acc)
        acc_v[...] = acc
        pltpu.sync_copy(acc_v, out_hbm.at[wid])

    partials = sc_kernel(sT, y, m_list, zT)
    return jnp.sum(partials) / jnp.float32(B)


# speedup vs baseline: 14.3464x; 1.0015x over previous
import functools

import jax
import jax.numpy as jnp
from jax import lax
from jax.experimental import pallas as pl
from jax.experimental.pallas import tpu as pltpu
from jax.experimental.pallas import tpu_sc as plsc

_K = 5
_NC = 2
_NS = 16
_NW = _NC * _NS
_L = 16
_LANES = 128
_NSTRIPES = 4
_NDB = 3


def _insert(ts, x):
    out = []
    for t in ts[:-1]:
        out.append(jnp.maximum(t, x))
        x = jnp.minimum(t, x)
    out.append(jnp.maximum(ts[-1], x))
    return out


def kernel(s, y, m_list, Z):
    B, D = s.shape
    S = Z.shape[2]
    sT = jnp.transpose(s)
    zT = jnp.transpose(Z, (2, 1, 0))
    rows_per_w = B // _NW

    mesh = plsc.VectorSubcoreMesh(core_axis_name="c", subcore_axis_name="s",
                                  num_cores=_NC, num_subcores=_NS)

    @functools.partial(
        pl.kernel,
        out_type=jax.ShapeDtypeStruct((_NW, _L), jnp.float32),
        mesh=mesh,
        scratch_types=[
            pltpu.VMEM((D,), jnp.float32),
            pltpu.VMEM((_L,), jnp.float32),
            pltpu.VMEM((D, _LANES), jnp.float32),
            pltpu.VMEM((_LANES,), jnp.int32),
            pltpu.VMEM((D, _LANES), jnp.float32),
            pltpu.VMEM((D, _LANES), jnp.float32),
        ] + [pltpu.VMEM((D, _LANES), jnp.float32) for _ in range(2 * _NDB)]
          + [pltpu.SemaphoreType.DMA] * 3,
        compiler_params=pltpu.CompilerParams(needs_layout_passes=False,
                                             use_tc_tiling_on_sc=True),
    )
    def sc_kernel(s_hbm, y_hbm, m_hbm, z_hbm, out_hbm, m_v, acc_v,
                  s_v, y_v, z3_v, z4_v,
                  za0, zb0, zc0, za1, zb1, zc1,
                  sem_a, sem_b, sem_s):
        slots = ((za0, zb0, zc0, sem_a), (za1, zb1, zc1, sem_b))
        cid = lax.axis_index("c")
        sid = lax.axis_index("s")
        wid = sid * _NC + cid
        w_base = wid * rows_per_w
        pltpu.sync_copy(m_hbm, m_v)

        def z03_copies(si, slot):
            b0 = w_base + si * _LANES
            sem = slot[3]
            return [pltpu.make_async_copy(z_hbm.at[j, :, pl.ds(b0, _LANES)],
                                          slot[j], sem) for j in range(_NDB)]

        def rest_copies(si):
            b0 = w_base + si * _LANES
            return [
                pltpu.make_async_copy(s_hbm.at[:, pl.ds(b0, _LANES)], s_v,
                                      sem_s),
                pltpu.make_async_copy(y_hbm.at[pl.ds(b0, _LANES)], y_v, sem_s),
                pltpu.make_async_copy(z_hbm.at[3, :, pl.ds(b0, _LANES)],
                                      z3_v, sem_s),
                pltpu.make_async_copy(z_hbm.at[4, :, pl.ds(b0, _LANES)],
                                      z4_v, sem_s),
            ]

        def compute_stripe(slot, acc):
            z_vs = (slot[0], slot[1], slot[2], z3_v, z4_v)

            def lg_body(lg, acc):
                l0 = lg * _L
                y_vec = y_v[pl.ds(l0, _L)]

                def step(d, ts, sy):
                    sval = s_v[d, pl.ds(l0, _L)]
                    eq = y_vec == d
                    basev = jnp.where(eq, jnp.float32(-1e10),
                                      sval * jnp.float32(500.0))
                    sy = sy + jnp.where(eq, sval, jnp.float32(0.0))
                    new = []
                    for j in range(S):
                        z = z_vs[j][d, pl.ds(l0, _L)]
                        new.extend(_insert(list(ts[j * _K:(j + 1) * _K]),
                                           basev + z))
                    return new, sy

                def dbody(i, carry):
                    ts = carry[:-1]
                    sy = carry[-1]
                    ts, sy = step(i * 2, ts, sy)
                    ts, sy = step(i * 2 + 1, ts, sy)
                    return tuple(ts) + (sy,)

                init = tuple(jnp.full((_L,), -3e38, jnp.float32)
                             for _ in range(_K * S))
                init = init + (jnp.zeros((_L,), jnp.float32),)
                res = lax.fori_loop(0, D // 2, dbody, init)
                ts, sy = res[:-1], res[-1]
                sum5 = (ts[_K - 1] + ts[2 * _K - 1] + ts[3 * _K - 1]
                        + ts[4 * _K - 1] + ts[5 * _K - 1])
                sigma = sum5 * jnp.float32(1.0 / (10.0 * S))
                mg = plsc.load_gather(m_v, [y_vec])
                s_y = (sy - mg) * jnp.float32(50.0)
                loss = jnp.maximum(jnp.float32(1.0) + sigma - s_y,
                                   jnp.float32(0.0))
                return acc + loss

            return lax.fori_loop(0, _LANES // _L, lg_body, acc)

        acc = jnp.zeros((_L,), jnp.float32)
        for c in z03_copies(0, slots[0]):
            c.start()

        def pair_body(i, acc):
            for k in range(2):
                si = i * 2 + k
                slot = slots[k]
                rest = rest_copies(si)
                for c in rest:
                    c.start()

                @pl.when(si + 1 < _NSTRIPES)
                def _():
                    for c in z03_copies(si + 1, slots[1 - k]):
                        c.start()

                for c in rest:
                    c.wait()
                for c in z03_copies(si, slot):
                    c.wait()
                acc = compute_stripe(slot, acc)
            return acc

        acc = lax.fori_loop(0, _NSTRIPES // 2, pair_body, acc)
        acc_v[...] = acc
        pltpu.sync_copy(acc_v, out_hbm.at[wid])

    partials = sc_kernel(sT, y, m_list, zT)
    return jnp.sum(partials) / jnp.float32(B)


# speedup vs baseline: 20.8900x; 1.4561x over previous
import functools

import jax
import jax.numpy as jnp
from jax import lax
from jax.experimental import pallas as pl
from jax.experimental.pallas import tpu as pltpu
from jax.experimental.pallas import tpu_sc as plsc

_K = 5
_NC = 2
_NS = 16
_NW = _NC * _NS
_L = 16
_LANES = 128
_NSTRIPES = 2
_NDB = 3
_BL = 512


def _merge_top5(a, b):
    r1 = jnp.maximum(a[0], b[0])
    r2 = jnp.maximum(jnp.maximum(a[1], b[1]), jnp.minimum(a[0], b[0]))
    r3 = jnp.maximum(jnp.maximum(a[2], b[2]),
                     jnp.maximum(jnp.minimum(a[0], b[1]),
                                 jnp.minimum(a[1], b[0])))
    r4 = jnp.maximum(jnp.maximum(a[3], b[3]),
                     jnp.maximum(jnp.minimum(a[0], b[2]),
                                 jnp.maximum(jnp.minimum(a[1], b[1]),
                                             jnp.minimum(a[2], b[0]))))
    r5 = jnp.maximum(jnp.maximum(a[4], b[4]),
                     jnp.maximum(
                         jnp.maximum(jnp.minimum(a[0], b[3]),
                                     jnp.minimum(a[1], b[2])),
                         jnp.maximum(jnp.minimum(a[2], b[1]),
                                     jnp.minimum(a[3], b[0]))))
    return [r1, r2, r3, r4, r5]


def _insert(ts, x):
    out = []
    for t in ts[:-1]:
        out.append(jnp.maximum(t, x))
        x = jnp.minimum(t, x)
    out.append(jnp.maximum(ts[-1], x))
    return out


def kernel(s, y, m_list, Z):
    B, D = s.shape
    S = Z.shape[2]
    sT = jnp.transpose(s)
    zT = jnp.transpose(Z, (2, 1, 0))
    rows_per_w = _NSTRIPES * _LANES
    b_sc = _NW * rows_per_w
    b_tc = B - b_sc

    mesh = plsc.VectorSubcoreMesh(core_axis_name="c", subcore_axis_name="s",
                                  num_cores=_NC, num_subcores=_NS)

    @functools.partial(
        pl.kernel,
        out_type=jax.ShapeDtypeStruct((_NW, _L), jnp.float32),
        mesh=mesh,
        scratch_types=[
            pltpu.VMEM((D,), jnp.float32),
            pltpu.VMEM((_L,), jnp.float32),
            pltpu.VMEM((D, _LANES), jnp.float32),
            pltpu.VMEM((_LANES,), jnp.int32),
            pltpu.VMEM((D, _LANES), jnp.float32),
            pltpu.VMEM((D, _LANES), jnp.float32),
        ] + [pltpu.VMEM((D, _LANES), jnp.float32) for _ in range(2 * _NDB)]
          + [pltpu.SemaphoreType.DMA] * 3,
        compiler_params=pltpu.CompilerParams(needs_layout_passes=False,
                                             use_tc_tiling_on_sc=True),
    )
    def sc_kernel(s_hbm, y_hbm, m_hbm, z_hbm, out_hbm, m_v, acc_v,
                  s_v, y_v, z3_v, z4_v,
                  za0, zb0, zc0, za1, zb1, zc1,
                  sem_a, sem_b, sem_s):
        slots = ((za0, zb0, zc0, sem_a), (za1, zb1, zc1, sem_b))
        cid = lax.axis_index("c")
        sid = lax.axis_index("s")
        wid = sid * _NC + cid
        w_base = wid * rows_per_w
        pltpu.sync_copy(m_hbm, m_v)

        def z03_copies(si, slot):
            b0 = w_base + si * _LANES
            sem = slot[3]
            return [pltpu.make_async_copy(z_hbm.at[j, :, pl.ds(b0, _LANES)],
                                          slot[j], sem) for j in range(_NDB)]

        def rest_copies(si):
            b0 = w_base + si * _LANES
            return [
                pltpu.make_async_copy(s_hbm.at[:, pl.ds(b0, _LANES)], s_v,
                                      sem_s),
                pltpu.make_async_copy(y_hbm.at[pl.ds(b0, _LANES)], y_v, sem_s),
                pltpu.make_async_copy(z_hbm.at[3, :, pl.ds(b0, _LANES)],
                                      z3_v, sem_s),
                pltpu.make_async_copy(z_hbm.at[4, :, pl.ds(b0, _LANES)],
                                      z4_v, sem_s),
            ]

        def compute_stripe(slot, acc):
            z_vs = (slot[0], slot[1], slot[2], z3_v, z4_v)

            def lg_body(lg, acc):
                l0 = lg * _L
                y_vec = y_v[pl.ds(l0, _L)]

                def step(d, ts, sy):
                    sval = s_v[d, pl.ds(l0, _L)]
                    eq = y_vec == d
                    basev = jnp.where(eq, jnp.float32(-1e10),
                                      sval * jnp.float32(500.0))
                    sy = sy + jnp.where(eq, sval, jnp.float32(0.0))
                    new = []
                    for j in range(S):
                        z = z_vs[j][d, pl.ds(l0, _L)]
                        new.extend(_insert(list(ts[j * _K:(j + 1) * _K]),
                                           basev + z))
                    return new, sy

                def dbody(i, carry):
                    ts = carry[:-1]
                    sy = carry[-1]
                    ts, sy = step(i * 2, ts, sy)
                    ts, sy = step(i * 2 + 1, ts, sy)
                    return tuple(ts) + (sy,)

                init = tuple(jnp.full((_L,), -3e38, jnp.float32)
                             for _ in range(_K * S))
                init = init + (jnp.zeros((_L,), jnp.float32),)
                res = lax.fori_loop(0, D // 2, dbody, init)
                ts, sy = res[:-1], res[-1]
                sum5 = (ts[_K - 1] + ts[2 * _K - 1] + ts[3 * _K - 1]
                        + ts[4 * _K - 1] + ts[5 * _K - 1])
                sigma = sum5 * jnp.float32(1.0 / (10.0 * S))
                mg = plsc.load_gather(m_v, [y_vec])
                s_y = (sy - mg) * jnp.float32(50.0)
                loss = jnp.maximum(jnp.float32(1.0) + sigma - s_y,
                                   jnp.float32(0.0))
                return acc + loss

            return lax.fori_loop(0, _LANES // _L, lg_body, acc)

        acc = jnp.zeros((_L,), jnp.float32)
        for c in z03_copies(0, slots[0]):
            c.start()

        def pair_body(i, acc):
            for k in range(2):
                si = i * 2 + k
                slot = slots[k]
                rest = rest_copies(si)
                for c in rest:
                    c.start()

                @pl.when(si + 1 < _NSTRIPES)
                def _():
                    for c in z03_copies(si + 1, slots[1 - k]):
                        c.start()

                for c in rest:
                    c.wait()
                for c in z03_copies(si, slot):
                    c.wait()
                acc = compute_stripe(slot, acc)
            return acc

        acc = lax.fori_loop(0, _NSTRIPES // 2, pair_body, acc)
        acc_v[...] = acc
        pltpu.sync_copy(acc_v, out_hbm.at[wid])

    def tc_body(s_ref, y_ref, m_ref, z_ref, out_ref):
        y_row = y_ref[0, :]
        sv = s_ref[...]
        dio = lax.broadcasted_iota(jnp.int32, (D, _BL), 0)
        eq = dio == y_row[None, :]
        base = jnp.where(eq, jnp.float32(-1e10), sv * jnp.float32(500.0))
        sy = jnp.sum(jnp.where(eq, sv, jnp.float32(0.0)), axis=0)
        my = jnp.sum(jnp.where(eq, m_ref[...], jnp.float32(0.0)), axis=0)
        slab_iota = lax.broadcasted_iota(jnp.int32, (8, _BL), 0)
        sigma_acc = jnp.zeros((_BL,), jnp.float32)
        for j in range(5):
            pert = base + z_ref[j]
            ts = [jnp.full((8, _BL), -3e38, jnp.float32) for _ in range(_K)]
            for k in range(D // 8):
                ts = _insert(ts, pert[k * 8:(k + 1) * 8, :])
            tail = pert[D - 8:D, :]
            tail = jnp.where(slab_iota < (8 - D % 8), jnp.float32(-3e38),
                             tail)
            ts = _insert(ts, tail)
            for sh in (4, 2, 1):
                ts = _merge_top5(ts, [pltpu.roll(t, sh, 0) for t in ts])
            sigma_acc = sigma_acc + ts[_K - 1][0, :]
        sigma = sigma_acc * jnp.float32(1.0 / 50.0)
        s_y = (sy - my) * jnp.float32(50.0)
        loss = jnp.maximum(jnp.float32(1.0) + sigma - s_y, jnp.float32(0.0))
        out_ref[...] = loss[None, :]

    y2 = y.reshape(1, B)
    m_col = m_list[:, None]
    base_blk = b_sc // _BL
    tc_losses = pl.pallas_call(
        tc_body,
        grid=(b_tc // _BL,),
        in_specs=[
            pl.BlockSpec((D, _BL), lambda i: (0, base_blk + i)),
            pl.BlockSpec((1, _BL), lambda i: (0, base_blk + i)),
            pl.BlockSpec((D, 1), lambda i: (0, 0)),
            pl.BlockSpec((5, D, _BL), lambda i: (0, 0, base_blk + i)),
        ],
        out_specs=pl.BlockSpec((1, _BL), lambda i: (0, i)),
        out_shape=jax.ShapeDtypeStruct((1, b_tc), jnp.float32),
    )(sT, y2, m_col, zT)

    partials = sc_kernel(sT, y, m_list, zT)
    return (jnp.sum(partials) + jnp.sum(tc_losses)) / jnp.float32(B)


# speedup vs baseline: 22.0866x; 1.0573x over previous
import functools

import jax
import jax.numpy as jnp
from jax import lax
from jax.experimental import pallas as pl
from jax.experimental.pallas import tpu as pltpu
from jax.experimental.pallas import tpu_sc as plsc

_K = 5
_NC = 2
_NS = 16
_NW = _NC * _NS
_L = 16
_LANES = 128
_NSTRIPES = 1
_NDB = 3
_BL = 512


def _merge_top5(a, b):
    r1 = jnp.maximum(a[0], b[0])
    r2 = jnp.maximum(jnp.maximum(a[1], b[1]), jnp.minimum(a[0], b[0]))
    r3 = jnp.maximum(jnp.maximum(a[2], b[2]),
                     jnp.maximum(jnp.minimum(a[0], b[1]),
                                 jnp.minimum(a[1], b[0])))
    r4 = jnp.maximum(jnp.maximum(a[3], b[3]),
                     jnp.maximum(jnp.minimum(a[0], b[2]),
                                 jnp.maximum(jnp.minimum(a[1], b[1]),
                                             jnp.minimum(a[2], b[0]))))
    r5 = jnp.maximum(jnp.maximum(a[4], b[4]),
                     jnp.maximum(
                         jnp.maximum(jnp.minimum(a[0], b[3]),
                                     jnp.minimum(a[1], b[2])),
                         jnp.maximum(jnp.minimum(a[2], b[1]),
                                     jnp.minimum(a[3], b[0]))))
    return [r1, r2, r3, r4, r5]


def _insert(ts, x):
    out = []
    for t in ts[:-1]:
        out.append(jnp.maximum(t, x))
        x = jnp.minimum(t, x)
    out.append(jnp.maximum(ts[-1], x))
    return out


def kernel(s, y, m_list, Z):
    B, D = s.shape
    S = Z.shape[2]
    sT = jnp.transpose(s)
    zT = jnp.transpose(Z, (2, 1, 0))
    rows_per_w = _NSTRIPES * _LANES
    b_sc = _NW * rows_per_w
    b_tc = B - b_sc

    mesh = plsc.VectorSubcoreMesh(core_axis_name="c", subcore_axis_name="s",
                                  num_cores=_NC, num_subcores=_NS)

    @functools.partial(
        pl.kernel,
        out_type=jax.ShapeDtypeStruct((_NW, _L), jnp.float32),
        mesh=mesh,
        scratch_types=[
            pltpu.VMEM((D,), jnp.float32),
            pltpu.VMEM((_L,), jnp.float32),
            pltpu.VMEM((D, _LANES), jnp.float32),
            pltpu.VMEM((_LANES,), jnp.int32),
            pltpu.VMEM((D, _LANES), jnp.float32),
            pltpu.VMEM((D, _LANES), jnp.float32),
        ] + [pltpu.VMEM((D, _LANES), jnp.float32) for _ in range(2 * _NDB)]
          + [pltpu.SemaphoreType.DMA] * 3,
        compiler_params=pltpu.CompilerParams(needs_layout_passes=False,
                                             use_tc_tiling_on_sc=True),
    )
    def sc_kernel(s_hbm, y_hbm, m_hbm, z_hbm, out_hbm, m_v, acc_v,
                  s_v, y_v, z3_v, z4_v,
                  za0, zb0, zc0, za1, zb1, zc1,
                  sem_a, sem_b, sem_s):
        slots = ((za0, zb0, zc0, sem_a), (za1, zb1, zc1, sem_b))
        cid = lax.axis_index("c")
        sid = lax.axis_index("s")
        wid = sid * _NC + cid
        w_base = wid * rows_per_w
        pltpu.sync_copy(m_hbm, m_v)

        def z03_copies(si, slot):
            b0 = w_base + si * _LANES
            sem = slot[3]
            return [pltpu.make_async_copy(z_hbm.at[j, :, pl.ds(b0, _LANES)],
                                          slot[j], sem) for j in range(_NDB)]

        def rest_copies(si):
            b0 = w_base + si * _LANES
            return [
                pltpu.make_async_copy(s_hbm.at[:, pl.ds(b0, _LANES)], s_v,
                                      sem_s),
                pltpu.make_async_copy(y_hbm.at[pl.ds(b0, _LANES)], y_v, sem_s),
                pltpu.make_async_copy(z_hbm.at[3, :, pl.ds(b0, _LANES)],
                                      z3_v, sem_s),
                pltpu.make_async_copy(z_hbm.at[4, :, pl.ds(b0, _LANES)],
                                      z4_v, sem_s),
            ]

        def compute_stripe(slot, acc):
            z_vs = (slot[0], slot[1], slot[2], z3_v, z4_v)

            def lg_body(lg, acc):
                l0 = lg * _L
                y_vec = y_v[pl.ds(l0, _L)]

                def step(d, ts, sy):
                    sval = s_v[d, pl.ds(l0, _L)]
                    eq = y_vec == d
                    basev = jnp.where(eq, jnp.float32(-1e10),
                                      sval * jnp.float32(500.0))
                    sy = sy + jnp.where(eq, sval, jnp.float32(0.0))
                    new = []
                    for j in range(S):
                        z = z_vs[j][d, pl.ds(l0, _L)]
                        new.extend(_insert(list(ts[j * _K:(j + 1) * _K]),
                                           basev + z))
                    return new, sy

                def dbody(i, carry):
                    ts = carry[:-1]
                    sy = carry[-1]
                    ts, sy = step(i * 2, ts, sy)
                    ts, sy = step(i * 2 + 1, ts, sy)
                    return tuple(ts) + (sy,)

                init = tuple(jnp.full((_L,), -3e38, jnp.float32)
                             for _ in range(_K * S))
                init = init + (jnp.zeros((_L,), jnp.float32),)
                res = lax.fori_loop(0, D // 2, dbody, init)
                ts, sy = res[:-1], res[-1]
                sum5 = (ts[_K - 1] + ts[2 * _K - 1] + ts[3 * _K - 1]
                        + ts[4 * _K - 1] + ts[5 * _K - 1])
                sigma = sum5 * jnp.float32(1.0 / (10.0 * S))
                mg = plsc.load_gather(m_v, [y_vec])
                s_y = (sy - mg) * jnp.float32(50.0)
                loss = jnp.maximum(jnp.float32(1.0) + sigma - s_y,
                                   jnp.float32(0.0))
                return acc + loss

            return lax.fori_loop(0, _LANES // _L, lg_body, acc)

        acc = jnp.zeros((_L,), jnp.float32)
        for c in z03_copies(0, slots[0]):
            c.start()
        for si in range(_NSTRIPES):
            slot = slots[si % 2]
            rest = rest_copies(si)
            for c in rest:
                c.start()
            if si + 1 < _NSTRIPES:
                for c in z03_copies(si + 1, slots[(si + 1) % 2]):
                    c.start()
            for c in rest:
                c.wait()
            for c in z03_copies(si, slot):
                c.wait()
            acc = compute_stripe(slot, acc)
        acc_v[...] = acc
        pltpu.sync_copy(acc_v, out_hbm.at[wid])

    def tc_body(s_ref, y_ref, m_ref, z_ref, out_ref):
        y_row = y_ref[0, :]
        sv = s_ref[...]
        dio = lax.broadcasted_iota(jnp.int32, (D, _BL), 0)
        eq = dio == y_row[None, :]
        base = jnp.where(eq, jnp.float32(-1e10), sv * jnp.float32(500.0))
        sy = jnp.sum(jnp.where(eq, sv, jnp.float32(0.0)), axis=0)
        my = jnp.sum(jnp.where(eq, m_ref[...], jnp.float32(0.0)), axis=0)
        slab_iota = lax.broadcasted_iota(jnp.int32, (8, _BL), 0)
        sigma_acc = jnp.zeros((_BL,), jnp.float32)
        for j in range(5):
            pert = base + z_ref[j]
            ts = [jnp.full((8, _BL), -3e38, jnp.float32) for _ in range(_K)]
            for k in range(D // 8):
                ts = _insert(ts, pert[k * 8:(k + 1) * 8, :])
            tail = pert[D - 8:D, :]
            tail = jnp.where(slab_iota < (8 - D % 8), jnp.float32(-3e38),
                             tail)
            ts = _insert(ts, tail)
            for sh in (4, 2, 1):
                ts = _merge_top5(ts, [pltpu.roll(t, sh, 0) for t in ts])
            sigma_acc = sigma_acc + ts[_K - 1][0, :]
        sigma = sigma_acc * jnp.float32(1.0 / 50.0)
        s_y = (sy - my) * jnp.float32(50.0)
        loss = jnp.maximum(jnp.float32(1.0) + sigma - s_y, jnp.float32(0.0))
        out_ref[...] = loss[None, :]

    y2 = y.reshape(1, B)
    m_col = m_list[:, None]
    base_blk = b_sc // _BL
    tc_losses = pl.pallas_call(
        tc_body,
        grid=(b_tc // _BL,),
        in_specs=[
            pl.BlockSpec((D, _BL), lambda i: (0, base_blk + i)),
            pl.BlockSpec((1, _BL), lambda i: (0, base_blk + i)),
            pl.BlockSpec((D, 1), lambda i: (0, 0)),
            pl.BlockSpec((5, D, _BL), lambda i: (0, 0, base_blk + i)),
        ],
        out_specs=pl.BlockSpec((1, _BL), lambda i: (0, i)),
        out_shape=jax.ShapeDtypeStruct((1, b_tc), jnp.float32),
    )(sT, y2, m_col, zT)

    partials = sc_kernel(sT, y, m_list, zT)
    return (jnp.sum(partials) + jnp.sum(tc_losses)) / jnp.float32(B)
